# baseline (speedup 1.0000x reference)
import functools

import jax
import jax.numpy as jnp
from jax import lax
from jax.experimental import pallas as pl
from jax.experimental.pallas import tpu as pltpu
from jax.experimental.pallas import tpu_sc as plsc

VOCAB = 1000000
EMBED_DIM = 64

NC = 2
NS = 16
NW = NC * NS

CHUNK = 128
K = 4
GROUP = CHUNK * K


def _lookup_kernel(n_groups, idx_hbm, table_hbm, out_hbm,
                   idx_v, buf_a, buf_b, sem_a, sem_b):
    wid = lax.axis_index("s") * NC + lax.axis_index("c")

    pltpu.sync_copy(idx_hbm.at[wid], idx_v)

    def fire(g, buf, sem):
        for k in range(K):
            pltpu.make_async_copy(
                table_hbm.at[idx_v.at[g * K + k]], buf.at[k], sem
            ).start()

    def drain(g, buf, sem):
        for k in range(K):
            pltpu.make_async_copy(
                table_hbm.at[idx_v.at[g * K + k]], buf.at[k], sem
            ).wait()

    fire(0, buf_a, sem_a)

    def body(i, _):
        g = 2 * i
        fire(g + 1, buf_b, sem_b)
        drain(g, buf_a, sem_a)
        pltpu.sync_copy(buf_a, out_hbm.at[wid].at[g])

        @pl.when(g + 2 < n_groups)
        def _():
            fire(g + 2, buf_a, sem_a)

        drain(g + 1, buf_b, sem_b)
        pltpu.sync_copy(buf_b, out_hbm.at[wid].at[g + 1])
        return 0

    lax.fori_loop(0, n_groups // 2, body, 0)


@jax.jit
def kernel(batch, table):
    B, L = batch.shape
    total = B * L
    assert total % (NW * GROUP) == 0
    per_w = total // NW
    n_groups = per_w // GROUP
    assert n_groups % 2 == 0

    idx = batch.reshape(NW, per_w // CHUNK, CHUNK).astype(jnp.int32)

    mesh = plsc.VectorSubcoreMesh(core_axis_name="c", subcore_axis_name="s")
    out = pl.kernel(
        functools.partial(_lookup_kernel, n_groups),
        out_type=jax.ShapeDtypeStruct((NW, n_groups, K, CHUNK, EMBED_DIM),
                                      jnp.float32),
        mesh=mesh,
        compiler_params=pltpu.CompilerParams(use_tc_tiling_on_sc=False),
        scratch_types=[
            pltpu.VMEM((per_w // CHUNK, CHUNK), jnp.int32),
            pltpu.VMEM((K, CHUNK, EMBED_DIM), jnp.float32),
            pltpu.VMEM((K, CHUNK, EMBED_DIM), jnp.float32),
            pltpu.SemaphoreType.DMA,
            pltpu.SemaphoreType.DMA,
        ],
    )(idx, table)

    return out.reshape(B, L, EMBED_DIM)
